# baseline (speedup 1.0000x reference)
import functools

import jax
import jax.numpy as jnp
from jax import lax
from jax.experimental import pallas as pl
from jax.experimental.pallas import tpu as pltpu
from jax.experimental.pallas import tpu_sc as plsc

_EPS = 1e-09
_C = 10
_NW = 16


@functools.cache
def _make_sc(B):
    cols = B // _NW
    mesh = plsc.VectorSubcoreMesh(
        core_axis_name="c", subcore_axis_name="s", num_cores=1
    )

    @functools.partial(
        pl.kernel,
        out_type=[
            jax.ShapeDtypeStruct((_C * B,), jnp.float32),
            jax.ShapeDtypeStruct((_C * B,), jnp.float32),
        ],
        mesh=mesh,
        compiler_params=pltpu.CompilerParams(
            needs_layout_passes=False,
            skip_device_barrier=True,
        ),
        scratch_types=[
            pltpu.VMEM((cols,), jnp.int32),
            pltpu.VMEM((_C * cols,), jnp.float32),
            pltpu.SemaphoreType.DMA,
            pltpu.SemaphoreType.DMA,
        ],
    )
    def k(labels_hbm, mu_hbm, std_hbm, lab_v, buf_v, lsem, osem):
        wid = lax.axis_index("s") + lax.axis_index("c") * 16
        cbase = wid * cols

        cp = pltpu.async_copy(labels_hbm.at[pl.ds(cbase, cols)], lab_v, lsem)

        eps16 = jnp.full((16,), _EPS, jnp.float32)

        def fill(i, carry):
            for j in range(_C):
                buf_v[pl.ds((i * _C + j) * 16, 16)] = eps16
            return carry

        lax.fori_loop(0, (_C * cols) // (16 * _C), fill, 0)

        std_cps = [
            pltpu.async_copy(
                buf_v.at[pl.ds(c * cols, cols)],
                std_hbm.at[pl.ds(c * B + cbase, cols)],
                osem,
            )
            for c in range(_C)
        ]
        cp.wait()
        for scp in std_cps:
            scp.wait()

        one16 = jnp.full((16,), jnp.float32(1.0 - _EPS), jnp.float32)
        iota16 = lax.iota(jnp.int32, 16)

        def scat(i, carry):
            lab = lab_v[pl.ds(i * 16, 16)]
            col = iota16 + i * 16
            plsc.store_scatter(buf_v, [lab * cols + col], one16)
            return carry

        lax.fori_loop(0, cols // 16, scat, 0)

        mu_cps = [
            pltpu.async_copy(
                buf_v.at[pl.ds(c * cols, cols)],
                mu_hbm.at[pl.ds(c * B + cbase, cols)],
                osem,
            )
            for c in range(_C)
        ]
        for mcp in mu_cps:
            mcp.wait()

    return k


def kernel(labels, cuda):
    B = labels.shape[0]
    mu_f, std_f = _make_sc(B)(labels)
    mu = jnp.transpose(mu_f.reshape(1, _C, B), (0, 2, 1))
    std = jnp.transpose(std_f.reshape(1, _C, B), (0, 2, 1))
    return mu, std
